# Initial kernel scaffold; baseline (speedup 1.0000x reference)
import jax
import jax.numpy as jnp
from jax.experimental import pallas as pl


def kernel(x, edge_index, edge_attr, batch, ln_gamma, ln_beta, c1_Wr, c1_br, c1_Wn, p1, c2_Wr, c2_br, c2_Wn, p2, c3_Wr, c3_br, c3_Wn, p3, o1_W, o1_b, o2_W, o2_b, o3_W, o3_b):
    raise NotImplementedError("write your pallas kernel here")


# speedup vs baseline: 2.5112x; 2.5112x over previous
import functools
import jax
import jax.numpy as jnp
from jax import lax
from jax.experimental import pallas as pl
from jax.experimental.pallas import tpu as pltpu
from jax.experimental.pallas import tpu_sc as plsc

N_NODES = 10000
NPAD = 10240
NBLK = 80
F = 16
NG = 16
E = 320000
EPT = E // 16
EPTP = 20096
NCH = EPTP // 128
ETOT = 16 * EPTP
FC = 5
YW = 6
NEG_INF = float("-inf")


def _pre_body(x_ref, g_ref, be_ref, wr_ref, wn_ref, y_ref, r_ref):
    xv = x_ref[...]
    mu = jnp.mean(xv, axis=1, keepdims=True)
    var = jnp.mean((xv - mu) * (xv - mu), axis=1, keepdims=True)
    ln = (xv - mu) / jnp.sqrt(var + 1e-5) * g_ref[...] + be_ref[...]
    y_ref[...] = jnp.dot(ln, wr_ref[...], preferred_element_type=jnp.float32)
    r_ref[...] = jnp.dot(ln, wn_ref[...], preferred_element_type=jnp.float32)


def _k_pre(xp, g, be, wr, wn):
    return pl.pallas_call(
        _pre_body,
        grid=(10,),
        in_specs=[
            pl.BlockSpec((1024, 128), lambda i: (i, 0)),
            pl.BlockSpec((1, 128), lambda i: (0, 0)),
            pl.BlockSpec((1, 128), lambda i: (0, 0)),
            pl.BlockSpec((128, F), lambda i: (0, 0)),
            pl.BlockSpec((128, F), lambda i: (0, 0)),
        ],
        out_specs=[
            pl.BlockSpec((1024, F), lambda i: (i, 0)),
            pl.BlockSpec((1024, F), lambda i: (i, 0)),
        ],
        out_shape=[
            jax.ShapeDtypeStruct((NPAD, F), jnp.float32),
            jax.ShapeDtypeStruct((NPAD, F), jnp.float32),
        ],
    )(xp, g, be, wr, wn)


def _hs_body(agg_ref, r_ref, br_ref, p_ref, h_ref, s_ref):
    a = jnp.sum(agg_ref[...], axis=0)
    h = jnp.maximum(a + br_ref[...] + r_ref[...], 0.0)
    p = p_ref[...]
    pn = jnp.sqrt(jnp.sum(p * p))
    s = jnp.tanh(jnp.dot(h, p.reshape(F, 1),
                         preferred_element_type=jnp.float32) / pn)
    h_ref[...] = h
    s_ref[...] = s


def _k_hs(agg, r, br, p):
    return pl.pallas_call(
        _hs_body,
        grid=(NBLK,),
        in_specs=[
            pl.BlockSpec((32, 128, F), lambda i: (0, i, 0)),
            pl.BlockSpec((128, F), lambda i: (i, 0)),
            pl.BlockSpec((1, F), lambda i: (0, 0)),
            pl.BlockSpec((1, F), lambda i: (0, 0)),
        ],
        out_specs=[
            pl.BlockSpec((128, F), lambda i: (i, 0)),
            pl.BlockSpec((128, 1), lambda i: (i, 0)),
        ],
        out_shape=[
            jax.ShapeDtypeStruct((NPAD, F), jnp.float32),
            jax.ShapeDtypeStruct((NPAD, 1), jnp.float32),
        ],
    )(agg, r, br, p)


def _rank_body(srow_ref, brow_ref, trow_ref, scol_ref, bcol_ref, tcol_ref,
               h_ref, wr_ref, wn_ref,
               kf_ref, bn_ref, tn_ref, y_ref, r_ref, x_ref,
               mx_ref, sm_ref, cnt_ref):
    i = pl.program_id(0)
    bi = bcol_ref[...]
    ti = tcol_ref[...]
    si = scol_ref[...]
    h = h_ref[...]
    brow = brow_ref[...]

    kb = jnp.zeros((128, 1), jnp.float32)
    for g in range(NG):
        cg = jnp.sum(jnp.where(brow == float(g), 1.0, 0.0))
        kg = jnp.floor((9.0 * cg + 9.0) / 10.0)
        kb = kb + jnp.where(bi == float(g), kg, 0.0)

    gmax_i = jnp.max(jnp.where(bi < 16.0, bi, -1.0))
    gmin_i = jnp.min(jnp.where(bi < 16.0, bi, 99.0))

    def jbody(j, rank):
        bj = brow_ref[pl.ds(j, 1), :]
        gmin_j = jnp.min(jnp.where(bj < 16.0, bj, 99.0))
        gmax_j = jnp.max(jnp.where(bj < 16.0, bj, -1.0))
        ok = (gmin_j <= gmax_i) & (gmax_j >= gmin_i)

        def do(r):
            sj = srow_ref[pl.ds(j, 1), :]
            tj = trow_ref[pl.ds(j, 1), :]
            cmp = (bj == bi) & ((sj > si) | ((sj == si) & (tj < ti)))
            return r + jnp.sum(jnp.where(cmp, 1.0, 0.0), axis=1, keepdims=True)

        return lax.cond(ok, do, lambda r: r, rank)

    rank = lax.fori_loop(0, NBLK, jbody, jnp.zeros((128, 1), jnp.float32))

    keep = (rank < kb) & (bi < 16.0)
    kf = jnp.where(keep, 1.0, 0.0)
    hp = h * si * kf
    bn = jnp.where(keep, bi, 16.0)

    kf_ref[...] = kf
    bn_ref[...] = bn
    tn_ref[...] = rank
    y_ref[...] = jnp.dot(hp, wr_ref[...], preferred_element_type=jnp.float32)
    r_ref[...] = jnp.dot(hp, wn_ref[...], preferred_element_type=jnp.float32)

    @pl.when(i == 0)
    def _():
        mx_ref[...] = jnp.full((NG, F), NEG_INF, jnp.float32)
        sm_ref[...] = jnp.zeros((NG, F), jnp.float32)
        cnt_ref[...] = jnp.zeros((NG, F), jnp.float32)

    for g in range(NG):
        m = bn == float(g)
        hm = jnp.where(m, hp, NEG_INF)
        hz = jnp.where(m, hp, 0.0)
        mx_ref[pl.ds(g, 1), :] = jnp.maximum(
            mx_ref[pl.ds(g, 1), :], jnp.max(hm, axis=0, keepdims=True))
        sm_ref[pl.ds(g, 1), :] = sm_ref[pl.ds(g, 1), :] + jnp.sum(
            hz, axis=0, keepdims=True)
        cnt_ref[pl.ds(g, 1), :] = cnt_ref[pl.ds(g, 1), :] + jnp.sum(
            jnp.where(m, 1.0, 0.0))

    @pl.when(i == NBLK - 1)
    def _():
        mean = sm_ref[...] / jnp.maximum(cnt_ref[...], 1.0)
        x_ref[...] = jnp.concatenate([mx_ref[...], mean], axis=1)


def _k_rank(srow, brow, trow, scol, bcol, tcol, h, wr, wn):
    return pl.pallas_call(
        _rank_body,
        grid=(NBLK,),
        in_specs=[
            pl.BlockSpec((NBLK, 128), lambda i: (0, 0)),
            pl.BlockSpec((NBLK, 128), lambda i: (0, 0)),
            pl.BlockSpec((NBLK, 128), lambda i: (0, 0)),
            pl.BlockSpec((128, 1), lambda i: (i, 0)),
            pl.BlockSpec((128, 1), lambda i: (i, 0)),
            pl.BlockSpec((128, 1), lambda i: (i, 0)),
            pl.BlockSpec((128, F), lambda i: (i, 0)),
            pl.BlockSpec((F, F), lambda i: (0, 0)),
            pl.BlockSpec((F, F), lambda i: (0, 0)),
        ],
        out_specs=[
            pl.BlockSpec((128, 1), lambda i: (i, 0)),
            pl.BlockSpec((128, 1), lambda i: (i, 0)),
            pl.BlockSpec((128, 1), lambda i: (i, 0)),
            pl.BlockSpec((128, F), lambda i: (i, 0)),
            pl.BlockSpec((128, F), lambda i: (i, 0)),
            pl.BlockSpec((NG, 2 * F), lambda i: (0, 0)),
        ],
        out_shape=[
            jax.ShapeDtypeStruct((NPAD, 1), jnp.float32),
            jax.ShapeDtypeStruct((NPAD, 1), jnp.float32),
            jax.ShapeDtypeStruct((NPAD, 1), jnp.float32),
            jax.ShapeDtypeStruct((NPAD, F), jnp.float32),
            jax.ShapeDtypeStruct((NPAD, F), jnp.float32),
            jax.ShapeDtypeStruct((NG, 2 * F), jnp.float32),
        ],
        scratch_shapes=[
            pltpu.VMEM((NG, F), jnp.float32),
            pltpu.VMEM((NG, F), jnp.float32),
            pltpu.VMEM((NG, F), jnp.float32),
        ],
    )(srow, brow, trow, scol, bcol, tcol, h, wr, wn)


def _out_body(x1_ref, x2_ref, x3_ref, w1_ref, b1_ref, w2_ref, b2_ref,
              w3_ref, b3_ref, o_ref):
    z = jnp.maximum(x1_ref[...] + x2_ref[...] + x3_ref[...], 0.0)
    z = jnp.concatenate([z, jnp.zeros((NG, 96), jnp.float32)], axis=1)
    z = jnp.maximum(
        jnp.dot(z, w1_ref[...], preferred_element_type=jnp.float32)
        + b1_ref[...], 0.0)
    z = jnp.maximum(
        jnp.dot(z, w2_ref[...], preferred_element_type=jnp.float32)
        + b2_ref[...], 0.0)
    z = jnp.dot(z, w3_ref[...], preferred_element_type=jnp.float32) + b3_ref[...]
    o_ref[...] = z[:, :16]


def _k_out(x1, x2, x3, w1, b1, w2, b2, w3, b3):
    return pl.pallas_call(
        _out_body,
        out_shape=jax.ShapeDtypeStruct((NG, 16), jnp.float32),
    )(x1, x2, x3, w1, b1, w2, b2, w3, b3)


def _sc_body(y_hbm, src_hbm, dst_hbm, ew_hbm,
             agg_hbm, ewo_hbm,
             y_v, src_v, dst_v, ew_v, ewo_v, acc_v):
    cid = lax.axis_index("c")
    sid = lax.axis_index("s")
    z16 = jnp.zeros((16,), jnp.float32)

    pltpu.sync_copy(y_hbm.at[cid], y_v)

    def zacc(r, c):
        acc_v[pl.ds(r * 16, 16)] = z16
        return c
    lax.fori_loop(0, NPAD * FC // 16, zacc, 0)

    base = sid * EPTP

    def chunk(c, carry):
        off = base + c * 128
        pltpu.sync_copy(src_hbm.at[pl.ds(off, 128)], src_v)
        pltpu.sync_copy(dst_hbm.at[pl.ds(off, 128)], dst_v)
        pltpu.sync_copy(ew_hbm.at[pl.ds(off, 128)], ew_v)

        def grp(g, cc):
            nidx = src_v[pl.ds(g * 16, 16)]
            didx = dst_v[pl.ds(g * 16, 16)]
            nb = nidx * YW
            db = didx * YW
            sk = plsc.load_gather(y_v, [nb + FC])
            dk = plsc.load_gather(y_v, [db + FC])
            w = ew_v[pl.ds(g * 16, 16)] * sk * dk
            ewo_v[pl.ds(g * 16, 16)] = w
            da = didx * FC
            for f in range(FC):
                vals = plsc.load_gather(y_v, [nb + f]) * w
                plsc.addupdate_scatter(acc_v, [da + f], vals)
            return cc
        lax.fori_loop(0, 8, grp, 0)

        @pl.when(cid == 0)
        def _():
            pltpu.sync_copy(ewo_v, ewo_hbm.at[pl.ds(off, 128)])
        return carry

    lax.fori_loop(0, NCH, chunk, 0)

    pltpu.sync_copy(acc_v, agg_hbm.at[pl.ds((cid * 16 + sid) * NPAD * FC,
                                            NPAD * FC)])


def _edge_aggregate(y10, kf, src, dst, ew):
    kfc = kf.reshape(NPAD, 1)
    y6 = jnp.stack([
        jnp.concatenate([y10[:, 0:FC], kfc], axis=1),
        jnp.concatenate([y10[:, FC:2 * FC], kfc], axis=1),
    ]).reshape(2, NPAD * YW)
    mesh = plsc.VectorSubcoreMesh(core_axis_name="c", subcore_axis_name="s")
    fn = pl.kernel(
        _sc_body,
        mesh=mesh,
        compiler_params=pltpu.CompilerParams(
            needs_layout_passes=False, use_tc_tiling_on_sc=False),
        out_type=[
            jax.ShapeDtypeStruct((32 * NPAD * FC,), jnp.float32),
            jax.ShapeDtypeStruct((ETOT,), jnp.float32),
        ],
        scratch_types=[
            pltpu.VMEM((NPAD * YW,), jnp.float32),
            pltpu.VMEM((128,), jnp.int32),
            pltpu.VMEM((128,), jnp.int32),
            pltpu.VMEM((128,), jnp.float32),
            pltpu.VMEM((128,), jnp.float32),
            pltpu.VMEM((NPAD * FC,), jnp.float32),
        ],
    )
    aggf, ewo = fn(y6, src, dst, ew)
    a = aggf.reshape(2, 16, NPAD, FC)
    a0 = jnp.pad(a[0], ((0, 0), (0, 0), (0, F - FC)))
    a1 = jnp.pad(a[1], ((0, 0), (0, 0), (FC, F - 2 * FC)))
    return jnp.concatenate([a0, a1], axis=0), ewo


def _padw(w, shape):
    out = jnp.zeros(shape, jnp.float32)
    return out.at[: w.shape[0], : w.shape[1]].set(w)


def kernel(x, edge_index, edge_attr, batch, ln_gamma, ln_beta, c1_Wr, c1_br,
           c1_Wn, p1, c2_Wr, c2_br, c2_Wn, p2, c3_Wr, c3_br, c3_Wn, p3,
           o1_W, o1_b, o2_W, o2_b, o3_W, o3_b):
    f32 = jnp.float32
    xp = jnp.pad(x, ((0, NPAD - N_NODES), (0, 0)))
    b0 = jnp.concatenate([batch.astype(f32), jnp.full((NPAD - N_NODES,), 16.0, f32)])
    t0 = jnp.arange(NPAD, dtype=f32)

    src = edge_index[0].reshape(16, EPT)
    dst = edge_index[1].reshape(16, EPT)
    ewe = edge_attr.astype(f32).reshape(16, EPT)
    pad_e = ((0, 0), (0, EPTP - EPT))
    srcp = jnp.pad(src, pad_e).reshape(-1)
    dstp = jnp.pad(dst, pad_e).reshape(-1)
    ewp = jnp.pad(ewe, pad_e).reshape(-1)

    g_row = ln_gamma.reshape(1, 128)
    be_row = ln_beta.reshape(1, 128)
    wr1 = _padw(c1_Wr.T, (128, F))
    wn1 = _padw(c1_Wn.T, (128, F))
    br1 = _padw(c1_br.reshape(1, -1), (1, F))
    p1r = _padw(p1.reshape(1, -1), (1, F))
    wr2 = _padw(c2_Wr.T, (F, F))
    wn2 = _padw(c2_Wn.T, (F, F))
    br2 = _padw(c2_br.reshape(1, -1), (1, F))
    p2r = _padw(p2.reshape(1, -1), (1, F))
    wr3 = _padw(c3_Wr.T, (F, F))
    wn3 = _padw(c3_Wn.T, (F, F))
    br3 = _padw(c3_br.reshape(1, -1), (1, F))
    p3r = _padw(p3.reshape(1, -1), (1, F))
    wz = jnp.zeros((F, F), f32)

    w1 = jnp.zeros((128, 128), f32)
    w1 = w1.at[0:10, 0:80].set(o1_W[:, 0:10].T)
    w1 = w1.at[16:26, 0:80].set(o1_W[:, 10:20].T)
    b1 = _padw(o1_b.reshape(1, -1), (1, 128))
    w2 = _padw(o2_W.T, (128, 320))
    b2 = o2_b.reshape(1, 320)
    w3 = _padw(o3_W.T, (320, 128))
    b3 = _padw(o3_b.reshape(1, -1), (1, 128))

    y, r = _k_pre(xp, g_row, be_row, wr1, wn1)
    kf = jnp.ones((NPAD,), f32)
    ew_cur = ewp
    b_col = b0.reshape(NPAD, 1)
    t_col = t0.reshape(NPAD, 1)
    xs = []
    for (br, pr, wrn, wnn) in ((br1, p1r, wr2, wn2),
                               (br2, p2r, wr3, wn3),
                               (br3, p3r, wz, wz)):
        agg, ew_cur = _edge_aggregate(y[:, :10], kf, srcp, dstp, ew_cur)
        h, s_col = _k_hs(agg, r, br, pr)
        kf_col, bn_col, tn_col, y, r, xr = _k_rank(
            s_col.reshape(NBLK, 128), b_col.reshape(NBLK, 128),
            t_col.reshape(NBLK, 128), s_col, b_col, t_col, h, wrn, wnn)
        kf = kf_col.reshape(NPAD)
        b_col, t_col = bn_col, tn_col
        xs.append(xr)

    return _k_out(xs[0], xs[1], xs[2], w1, b1, w2, b2, w3, b3)
